# Initial kernel scaffold; baseline (speedup 1.0000x reference)
import functools

import jax
import jax.numpy as jnp
from jax import lax
from jax.experimental import pallas as pl
from jax.experimental.pallas import tpu as pltpu
from jax.experimental.pallas import tpu_sc as plsc

VOCAB = 1000000
D = 64
B = 16384 * 50

NC = 2
NS = 16
NW = NC * NS
B_PER_W = B // NW
IDX_ROW = 128
N_ROWS = B_PER_W // IDX_ROW
K = 8
N_GROUPS = N_ROWS // K
GROUP = K * IDX_ROW


def _embed_body(x_hbm, table_hbm, out_hbm, idx_v, rows_v, sem):
    wid = lax.axis_index("s") * NC + lax.axis_index("c")
    pltpu.sync_copy(x_hbm.at[wid], idx_v)
    out_base = wid * B_PER_W

    @pl.loop(0, N_GROUPS)
    def _group(g):
        row0 = g * K
        copies = []
        for b in range(K):
            copies.append(
                pltpu.async_copy(
                    table_hbm.at[idx_v.at[row0 + b]],
                    rows_v.at[pl.ds(b * IDX_ROW, IDX_ROW)],
                    sem,
                )
            )
        for c in copies:
            c.wait()
        pltpu.sync_copy(rows_v, out_hbm.at[pl.ds(out_base + g * GROUP, GROUP)])


@jax.jit
def _embed(x_flat3, table):
    mesh = plsc.VectorSubcoreMesh(core_axis_name="c", subcore_axis_name="s")
    return pl.kernel(
        _embed_body,
        out_type=jax.ShapeDtypeStruct((B, D), jnp.float32),
        mesh=mesh,
        scratch_types=[
            pltpu.VMEM((N_ROWS, IDX_ROW), jnp.int32),
            pltpu.VMEM((GROUP, D), jnp.float32),
            pltpu.SemaphoreType.DMA,
        ],
    )(x_flat3, table)


def kernel(x, table):
    x_flat3 = x.reshape(NW, N_ROWS, IDX_ROW).astype(jnp.int32)
    out = _embed(x_flat3, table)
    return out.reshape(x.shape[0], x.shape[1], D)


# speedup vs baseline: 1.8580x; 1.8580x over previous
import functools

import jax
import jax.numpy as jnp
from jax import lax
from jax.experimental import pallas as pl
from jax.experimental.pallas import tpu as pltpu
from jax.experimental.pallas import tpu_sc as plsc

VOCAB = 1000000
D = 64
B = 16384 * 50

NC = 2
NS = 16
NW = NC * NS
B_PER_W = B // NW
IDX_ROW = 128
N_ROWS = B_PER_W // IDX_ROW
K = 8
N_GROUPS = N_ROWS // K
GROUP = K * IDX_ROW


def _embed_body(x_hbm, table_hbm, out_hbm, idx_v, rows_v, sem):
    wid = lax.axis_index("s") * NC + lax.axis_index("c")
    pltpu.sync_copy(x_hbm.at[wid], idx_v)
    out_base = wid * B_PER_W

    @pl.loop(0, N_GROUPS)
    def _group(g):
        row0 = g * K
        copies = []
        for b in range(K):
            copies.append(
                pltpu.async_copy(
                    table_hbm.at[idx_v.at[row0 + b]],
                    rows_v.at[pl.ds(b * IDX_ROW, IDX_ROW)],
                    sem,
                )
            )
        for c in copies:
            c.wait()
        pltpu.sync_copy(rows_v, out_hbm.at[pl.ds(out_base + g * GROUP, GROUP)])


@jax.jit
def _embed(x_flat3, table):
    mesh = plsc.VectorSubcoreMesh(core_axis_name="c", subcore_axis_name="s")
    return pl.kernel(
        _embed_body,
        out_type=jax.ShapeDtypeStruct((B, D), jnp.float32),
        mesh=mesh,
        compiler_params=pltpu.CompilerParams(use_tc_tiling_on_sc=False),
        scratch_types=[
            pltpu.VMEM((N_ROWS, IDX_ROW), jnp.int32),
            pltpu.VMEM((GROUP, D), jnp.float32),
            pltpu.SemaphoreType.DMA,
        ],
    )(x_flat3, table)


def kernel(x, table):
    x_flat3 = x.reshape(NW, N_ROWS, IDX_ROW).astype(jnp.int32)
    out = _embed(x_flat3, table)
    return out.reshape(x.shape[0], x.shape[1], D)


# speedup vs baseline: 1.8758x; 1.0096x over previous
import functools

import jax
import jax.numpy as jnp
from jax import lax
from jax.experimental import pallas as pl
from jax.experimental.pallas import tpu as pltpu
from jax.experimental.pallas import tpu_sc as plsc

VOCAB = 1000000
D = 64
B = 16384 * 50

NC = 2
NS = 16
NW = NC * NS
B_PER_W = B // NW
IDX_ROW = 128
N_ROWS = B_PER_W // IDX_ROW
K = 4
N_GROUPS = N_ROWS // K
GROUP = K * IDX_ROW


def _embed_body(x_hbm, table_hbm, out_hbm, idx_v, rows0, rows1,
                gsem0, gsem1, osem0, osem1):
    wid = lax.axis_index("s") * NC + lax.axis_index("c")
    pltpu.sync_copy(x_hbm.at[wid], idx_v)
    out_base = wid * B_PER_W
    bufs = (rows0, rows1)
    gsems = (gsem0, gsem1)
    osems = (osem0, osem1)

    def fire_g(g, buf):
        for b in range(K):
            pltpu.async_copy(
                table_hbm.at[idx_v.at[g * K + b]],
                bufs[buf].at[pl.ds(b * IDX_ROW, IDX_ROW)],
                gsems[buf],
            )

    def drain_g(g, buf):
        for b in range(K):
            pltpu.make_async_copy(
                table_hbm.at[idx_v.at[g * K + b]],
                bufs[buf].at[pl.ds(b * IDX_ROW, IDX_ROW)],
                gsems[buf],
            ).wait()

    def out_slice(g):
        return out_hbm.at[pl.ds(out_base + g * GROUP, GROUP)]

    def fire_out(g, buf):
        pltpu.async_copy(bufs[buf], out_slice(g), osems[buf])

    def wait_out(g, buf):
        pltpu.make_async_copy(bufs[buf], out_slice(g), osems[buf]).wait()

    fire_g(0, 0)
    fire_g(1, 1)
    drain_g(0, 0)
    fire_out(0, 0)

    @pl.loop(2, N_GROUPS, step=2)
    def _group(g0):
        wait_out(g0 - 2, 0)
        fire_g(g0, 0)
        drain_g(g0 - 1, 1)
        fire_out(g0 - 1, 1)
        wait_out(g0 - 1, 1)
        fire_g(g0 + 1, 1)
        drain_g(g0, 0)
        fire_out(g0, 0)

    drain_g(N_GROUPS - 1, 1)
    fire_out(N_GROUPS - 1, 1)
    wait_out(N_GROUPS - 2, 0)
    wait_out(N_GROUPS - 1, 1)


@jax.jit
def _embed(x_flat3, table):
    mesh = plsc.VectorSubcoreMesh(core_axis_name="c", subcore_axis_name="s")
    return pl.kernel(
        _embed_body,
        out_type=jax.ShapeDtypeStruct((B, D), jnp.float32),
        mesh=mesh,
        compiler_params=pltpu.CompilerParams(use_tc_tiling_on_sc=False),
        scratch_types=[
            pltpu.VMEM((N_ROWS, IDX_ROW), jnp.int32),
            pltpu.VMEM((GROUP, D), jnp.float32),
            pltpu.VMEM((GROUP, D), jnp.float32),
            pltpu.SemaphoreType.DMA,
            pltpu.SemaphoreType.DMA,
            pltpu.SemaphoreType.DMA,
            pltpu.SemaphoreType.DMA,
        ],
    )(x_flat3, table)


def kernel(x, table):
    x_flat3 = x.reshape(NW, N_ROWS, IDX_ROW).astype(jnp.int32)
    out = _embed(x_flat3, table)
    return out.reshape(x.shape[0], x.shape[1], D)
